# baseline (speedup 1.0000x reference)
import functools

import jax
import jax.numpy as jnp
from jax import lax
from jax.experimental import pallas as pl
from jax.experimental.pallas import tpu as pltpu
from jax.experimental.pallas import tpu_sc as plsc

_LBLK = 512


def _pack_body(n_lanes, a_ref, pe_ref, po_ref, out_ref):
    g = pl.program_id(0)
    lane_ids = g * _LBLK + lax.broadcasted_iota(
        jnp.int32, a_ref.shape, dimension=1)
    a = jnp.where(lane_ids < n_lanes, a_ref[...], 0.0)
    dn = (((1,), (1,)), ((), ()))
    left = lax.dot_general(pe_ref[...], a, dn,
                           preferred_element_type=jnp.float32)
    right = lax.dot_general(po_ref[...], a, dn,
                            preferred_element_type=jnp.float32)
    out_ref[...] = jnp.concatenate([left, right], axis=1)


def _pack_table(tab_t, p_even, p_odd):
    embed, n = tab_t.shape
    grid = ((n + _LBLK - 1) // _LBLK,)
    return pl.pallas_call(
        functools.partial(_pack_body, n),
        grid=grid,
        in_specs=[
            pl.BlockSpec((embed, _LBLK), lambda g: (0, g)),
            pl.BlockSpec((_LBLK // 2, _LBLK), lambda g: (0, 0)),
            pl.BlockSpec((_LBLK // 2, _LBLK), lambda g: (0, 0)),
        ],
        out_specs=pl.BlockSpec((_LBLK // 2, 2 * embed), lambda g: (g, 0)),
        out_shape=jax.ShapeDtypeStruct((n // 2, 2 * embed), jnp.float32),
    )(tab_t, p_even, p_odd)


def _sc_gather(ku, ki, upacked, ipacked):
    info = plsc.get_sparse_core_info()
    nw = info.num_cores * info.num_subcores
    batch = ku.shape[0]
    width = upacked.shape[1]
    b_per_w = batch // nw
    half = b_per_w // 2

    mesh = plsc.VectorSubcoreMesh(core_axis_name="c", subcore_axis_name="s")
    out_struct = jax.ShapeDtypeStruct((batch, width), jnp.float32)

    @functools.partial(
        pl.kernel,
        mesh=mesh,
        compiler_params=pltpu.CompilerParams(needs_layout_passes=False),
        out_type=[out_struct, out_struct],
        scratch_types=[
            pltpu.VMEM((b_per_w,), jnp.int32),
            pltpu.VMEM((b_per_w,), jnp.int32),
            pltpu.VMEM((half, 128), jnp.float32),
            pltpu.VMEM((half, 128), jnp.float32),
            pltpu.SemaphoreType.DMA,
            pltpu.SemaphoreType.DMA,
            pltpu.SemaphoreType.DMA,
            pltpu.SemaphoreType.DMA,
        ],
    )
    def gather_kernel(ku_hbm, ki_hbm, utab_hbm, itab_hbm,
                      outu_hbm, outi_hbm,
                      idx_u, idx_i, buf_u, buf_i,
                      sem_u, sem_i, sem_wu, sem_wi):
        wid = lax.axis_index("s") * info.num_cores + lax.axis_index("c")
        base = wid * b_per_w
        pltpu.sync_copy(ku_hbm.at[pl.ds(base, b_per_w)], idx_u)
        pltpu.sync_copy(ki_hbm.at[pl.ds(base, b_per_w)], idx_i)

        for h in range(2):
            osl = pl.ds(base + h * half, half)
            isl = pl.ds(h * half, half)
            gu = pltpu.async_copy(
                utab_hbm.at[idx_u.at[isl]], buf_u, sem_u)
            gi = pltpu.async_copy(
                itab_hbm.at[idx_i.at[isl]], buf_i, sem_i)
            gu.wait()
            wu = pltpu.async_copy(buf_u, outu_hbm.at[osl], sem_wu)
            gi.wait()
            wi = pltpu.async_copy(buf_i, outi_hbm.at[osl], sem_wi)
            wu.wait()
            wi.wait()

    return gather_kernel(ku, ki, upacked, ipacked)


def _mlp_body(xu_ref, xi_ref, mu_ref, mi_ref, w1u_ref, w1i_ref, b1_ref,
              w2_ref, b2_ref, out_ref):
    embed = w1u_ref.shape[1]
    xu2 = xu_ref[...]
    xi2 = xi_ref[...]
    mu = mu_ref[...]
    mi = mi_ref[...]
    xu = xu2[:, :embed] * (1.0 - mu) + xu2[:, embed:] * mu
    xi = xi2[:, :embed] * (1.0 - mi) + xi2[:, embed:] * mi
    dn = (((1,), (1,)), ((), ()))
    h = lax.dot_general(xu, w1u_ref[...], dn,
                        preferred_element_type=jnp.float32,
                        precision=lax.Precision.HIGHEST)
    h += lax.dot_general(xi, w1i_ref[...], dn,
                         preferred_element_type=jnp.float32,
                         precision=lax.Precision.HIGHEST)
    h = jnp.maximum(h + b1_ref[...], 0.0)
    out = jnp.sum(h * w2_ref[...], axis=1, keepdims=True)
    out_ref[...] = out + b2_ref[0, 0]


def _tc_mlp(xu2, xi2, mu, mi, W1, b1, W2, b2):
    batch, width = xu2.shape
    embed = width // 2
    hidden = W1.shape[0]
    w1u = W1[:, :embed]
    w1i = W1[:, embed:]
    b1r = b1.reshape(1, hidden)
    b2r = b2.reshape(1, 1)
    blk = 2048
    grid = (batch // blk,)
    return pl.pallas_call(
        _mlp_body,
        grid=grid,
        in_specs=[
            pl.BlockSpec((blk, width), lambda i: (i, 0)),
            pl.BlockSpec((blk, width), lambda i: (i, 0)),
            pl.BlockSpec((blk, embed), lambda i: (i, 0)),
            pl.BlockSpec((blk, embed), lambda i: (i, 0)),
            pl.BlockSpec((hidden, embed), lambda i: (0, 0)),
            pl.BlockSpec((hidden, embed), lambda i: (0, 0)),
            pl.BlockSpec((1, hidden), lambda i: (0, 0)),
            pl.BlockSpec((1, hidden), lambda i: (0, 0)),
            pl.BlockSpec((1, 1), lambda i: (0, 0)),
        ],
        out_specs=pl.BlockSpec((blk, 1), lambda i: (i, 0)),
        out_shape=jax.ShapeDtypeStruct((batch, 1), jnp.float32),
    )(xu2, xi2, mu, mi, w1u, w1i, b1r, W2, b2r)


@jax.jit
def kernel(users, items, user_table, item_table, W1, b1, W2, b2):
    batch = users.shape[0]
    embed = user_table.shape[1]
    lanes = jnp.arange(_LBLK, dtype=jnp.int32)
    rows = jnp.arange(_LBLK // 2, dtype=jnp.int32)
    p_even = (lanes[None, :] == 2 * rows[:, None]).astype(jnp.float32)
    p_odd = (lanes[None, :] == 2 * rows[:, None] + 1).astype(jnp.float32)
    upacked = _pack_table(user_table.T, p_even, p_odd)
    ipacked = _pack_table(item_table.T, p_even, p_odd)
    ku = users >> 1
    ki = items >> 1
    mu = jnp.broadcast_to(
        (users & 1).astype(jnp.float32)[:, None], (batch, embed))
    mi = jnp.broadcast_to(
        (items & 1).astype(jnp.float32)[:, None], (batch, embed))
    xu2, xi2 = _sc_gather(ku, ki, upacked, ipacked)
    return _tc_mlp(xu2, xi2, mu, mi, W1, b1, W2, b2)


# speedup vs baseline: 4.6080x; 4.6080x over previous
import functools

import jax
import jax.numpy as jnp
from jax import lax
from jax.experimental import pallas as pl
from jax.experimental.pallas import tpu as pltpu
from jax.experimental.pallas import tpu_sc as plsc

_LBLK = 8192


def _pack_body(a_ref, eye_ref, out_ref):
    dn = (((0,), (0,)), ((), ()))
    out_ref[...] = lax.dot_general(
        a_ref[...], eye_ref[...], dn, preferred_element_type=jnp.float32)


def _pad_transpose(tab_t, eye):
    embed, n = tab_t.shape
    grid = ((n + _LBLK - 1) // _LBLK,)
    return pl.pallas_call(
        _pack_body,
        grid=grid,
        in_specs=[
            pl.BlockSpec((embed, _LBLK), lambda g: (0, g)),
            pl.BlockSpec((embed, 2 * embed), lambda g: (0, 0)),
        ],
        out_specs=pl.BlockSpec((_LBLK, 2 * embed), lambda g: (g, 0)),
        out_shape=jax.ShapeDtypeStruct((n, 2 * embed), jnp.float32),
    )(tab_t, eye)


def _sc_gather(users, items, urows, irows):
    info = plsc.get_sparse_core_info()
    nw = info.num_cores * info.num_subcores
    batch = users.shape[0]
    width = urows.shape[1]
    b_per_w = batch // nw
    half = b_per_w // 2

    mesh = plsc.VectorSubcoreMesh(core_axis_name="c", subcore_axis_name="s")
    out_struct = jax.ShapeDtypeStruct((batch, width), jnp.float32)

    @functools.partial(
        pl.kernel,
        mesh=mesh,
        compiler_params=pltpu.CompilerParams(needs_layout_passes=False),
        out_type=[out_struct, out_struct],
        scratch_types=[
            pltpu.VMEM((b_per_w,), jnp.int32),
            pltpu.VMEM((b_per_w,), jnp.int32),
            pltpu.VMEM((half, 128), jnp.float32),
            pltpu.VMEM((half, 128), jnp.float32),
            pltpu.SemaphoreType.DMA,
            pltpu.SemaphoreType.DMA,
            pltpu.SemaphoreType.DMA,
            pltpu.SemaphoreType.DMA,
        ],
    )
    def gather_kernel(users_hbm, items_hbm, utab_hbm, itab_hbm,
                      outu_hbm, outi_hbm,
                      idx_u, idx_i, buf_u, buf_i,
                      sem_u, sem_i, sem_wu, sem_wi):
        wid = lax.axis_index("s") * info.num_cores + lax.axis_index("c")
        base = wid * b_per_w
        pltpu.sync_copy(users_hbm.at[pl.ds(base, b_per_w)], idx_u)
        pltpu.sync_copy(items_hbm.at[pl.ds(base, b_per_w)], idx_i)

        for h in range(2):
            osl = pl.ds(base + h * half, half)
            isl = pl.ds(h * half, half)
            gu = pltpu.async_copy(
                utab_hbm.at[idx_u.at[isl]], buf_u, sem_u)
            gi = pltpu.async_copy(
                itab_hbm.at[idx_i.at[isl]], buf_i, sem_i)
            gu.wait()
            wu = pltpu.async_copy(buf_u, outu_hbm.at[osl], sem_wu)
            gi.wait()
            wi = pltpu.async_copy(buf_i, outi_hbm.at[osl], sem_wi)
            wu.wait()
            wi.wait()

    return gather_kernel(users, items, urows, irows)


def _mlp_body(xu_ref, xi_ref, w1u_ref, w1i_ref, b1_ref, w2_ref, b2_ref,
              out_ref):
    embed = w1u_ref.shape[1]
    xu = xu_ref[:, :embed]
    xi = xi_ref[:, :embed]
    dn = (((1,), (1,)), ((), ()))
    h = lax.dot_general(xu, w1u_ref[...], dn,
                        preferred_element_type=jnp.float32,
                        precision=lax.Precision.HIGHEST)
    h += lax.dot_general(xi, w1i_ref[...], dn,
                         preferred_element_type=jnp.float32,
                         precision=lax.Precision.HIGHEST)
    h = jnp.maximum(h + b1_ref[...], 0.0)
    out = jnp.sum(h * w2_ref[...], axis=1, keepdims=True)
    out_ref[...] = out + b2_ref[0, 0]


def _tc_mlp(xu2, xi2, W1, b1, W2, b2):
    batch, width = xu2.shape
    embed = width // 2
    hidden = W1.shape[0]
    w1u = W1[:, :embed]
    w1i = W1[:, embed:]
    b1r = b1.reshape(1, hidden)
    b2r = b2.reshape(1, 1)
    blk = 2048
    grid = (batch // blk,)
    return pl.pallas_call(
        _mlp_body,
        grid=grid,
        in_specs=[
            pl.BlockSpec((blk, width), lambda i: (i, 0)),
            pl.BlockSpec((blk, width), lambda i: (i, 0)),
            pl.BlockSpec((hidden, embed), lambda i: (0, 0)),
            pl.BlockSpec((hidden, embed), lambda i: (0, 0)),
            pl.BlockSpec((1, hidden), lambda i: (0, 0)),
            pl.BlockSpec((1, hidden), lambda i: (0, 0)),
            pl.BlockSpec((1, 1), lambda i: (0, 0)),
        ],
        out_specs=pl.BlockSpec((blk, 1), lambda i: (i, 0)),
        out_shape=jax.ShapeDtypeStruct((batch, 1), jnp.float32),
    )(xu2, xi2, w1u, w1i, b1r, W2, b2r)


@jax.jit
def kernel(users, items, user_table, item_table, W1, b1, W2, b2):
    embed = user_table.shape[1]
    eye = jnp.concatenate(
        [jnp.eye(embed, dtype=jnp.float32),
         jnp.zeros((embed, embed), jnp.float32)], axis=1)
    urows = _pad_transpose(user_table.T, eye)
    irows = _pad_transpose(item_table.T, eye)
    xu2, xi2 = _sc_gather(users, items, urows, irows)
    return _tc_mlp(xu2, xi2, W1, b1, W2, b2)


# speedup vs baseline: 5.5528x; 1.2050x over previous
import functools

import jax
import jax.numpy as jnp
from jax import lax
from jax.experimental import pallas as pl
from jax.experimental.pallas import tpu as pltpu
from jax.experimental.pallas import tpu_sc as plsc

_LBLK = 8192


def _pack_body(a_ref, b_ref, eye_ref, out_ref):
    dn = (((0,), (0,)), ((), ()))
    left = lax.dot_general(
        a_ref[...], eye_ref[...], dn, preferred_element_type=jnp.float32)
    right = lax.dot_general(
        b_ref[...], eye_ref[...], dn, preferred_element_type=jnp.float32)
    out_ref[...] = jnp.concatenate([left, right], axis=1)


def _pack_pair(utab_t, itab_t, eye):
    embed, n = utab_t.shape
    grid = ((n + _LBLK - 1) // _LBLK,)
    return pl.pallas_call(
        _pack_body,
        grid=grid,
        in_specs=[
            pl.BlockSpec((embed, _LBLK), lambda g: (0, g)),
            pl.BlockSpec((embed, _LBLK), lambda g: (0, g)),
            pl.BlockSpec((embed, embed), lambda g: (0, 0)),
        ],
        out_specs=pl.BlockSpec((_LBLK, 2 * embed), lambda g: (g, 0)),
        out_shape=jax.ShapeDtypeStruct((n, 2 * embed), jnp.float32),
    )(utab_t, itab_t, eye)


def _sc_gather(users, items, urows, irows):
    info = plsc.get_sparse_core_info()
    nw = info.num_cores * info.num_subcores
    batch = users.shape[0]
    width = urows.shape[1]
    b_per_w = batch // nw
    half = b_per_w // 2

    mesh = plsc.VectorSubcoreMesh(core_axis_name="c", subcore_axis_name="s")
    out_struct = jax.ShapeDtypeStruct((batch, width), jnp.float32)

    @functools.partial(
        pl.kernel,
        mesh=mesh,
        compiler_params=pltpu.CompilerParams(needs_layout_passes=False),
        out_type=[out_struct, out_struct],
        scratch_types=[
            pltpu.VMEM((b_per_w,), jnp.int32),
            pltpu.VMEM((b_per_w,), jnp.int32),
            pltpu.VMEM((half, 128), jnp.float32),
            pltpu.VMEM((half, 128), jnp.float32),
            pltpu.SemaphoreType.DMA,
            pltpu.SemaphoreType.DMA,
            pltpu.SemaphoreType.DMA,
            pltpu.SemaphoreType.DMA,
        ],
    )
    def gather_kernel(users_hbm, items_hbm, utab_hbm, itab_hbm,
                      outu_hbm, outi_hbm,
                      idx_u, idx_i, buf_u, buf_i,
                      sem_u, sem_i, sem_wu, sem_wi):
        wid = lax.axis_index("s") * info.num_cores + lax.axis_index("c")
        base = wid * b_per_w
        pltpu.sync_copy(users_hbm.at[pl.ds(base, b_per_w)], idx_u)
        pltpu.sync_copy(items_hbm.at[pl.ds(base, b_per_w)], idx_i)

        for h in range(2):
            osl = pl.ds(base + h * half, half)
            isl = pl.ds(h * half, half)
            gu = pltpu.async_copy(
                utab_hbm.at[idx_u.at[isl]], buf_u, sem_u)
            gi = pltpu.async_copy(
                itab_hbm.at[idx_i.at[isl]], buf_i, sem_i)
            gu.wait()
            wu = pltpu.async_copy(buf_u, outu_hbm.at[osl], sem_wu)
            gi.wait()
            wi = pltpu.async_copy(buf_i, outi_hbm.at[osl], sem_wi)
            wu.wait()
            wi.wait()

    return gather_kernel(users, items, urows, irows)


def _mlp_body(xu_ref, xi_ref, w1u_ref, w1i_ref, b1_ref, w2_ref, b2_ref,
              out_ref):
    embed = w1u_ref.shape[1]
    xu = xu_ref[:, :embed]
    xi = xi_ref[:, embed:]
    dn = (((1,), (1,)), ((), ()))
    h = lax.dot_general(xu, w1u_ref[...], dn,
                        preferred_element_type=jnp.float32,
                        precision=lax.Precision.HIGHEST)
    h += lax.dot_general(xi, w1i_ref[...], dn,
                         preferred_element_type=jnp.float32,
                         precision=lax.Precision.HIGHEST)
    h = jnp.maximum(h + b1_ref[...], 0.0)
    out = jnp.sum(h * w2_ref[...], axis=1, keepdims=True)
    out_ref[...] = out + b2_ref[0, 0]


def _tc_mlp(xu2, xi2, W1, b1, W2, b2):
    batch, width = xu2.shape
    embed = width // 2
    hidden = W1.shape[0]
    w1u = W1[:, :embed]
    w1i = W1[:, embed:]
    b1r = b1.reshape(1, hidden)
    b2r = b2.reshape(1, 1)
    blk = 2048
    grid = (batch // blk,)
    return pl.pallas_call(
        _mlp_body,
        grid=grid,
        in_specs=[
            pl.BlockSpec((blk, width), lambda i: (i, 0)),
            pl.BlockSpec((blk, width), lambda i: (i, 0)),
            pl.BlockSpec((hidden, embed), lambda i: (0, 0)),
            pl.BlockSpec((hidden, embed), lambda i: (0, 0)),
            pl.BlockSpec((1, hidden), lambda i: (0, 0)),
            pl.BlockSpec((1, hidden), lambda i: (0, 0)),
            pl.BlockSpec((1, 1), lambda i: (0, 0)),
        ],
        out_specs=pl.BlockSpec((blk, 1), lambda i: (i, 0)),
        out_shape=jax.ShapeDtypeStruct((batch, 1), jnp.float32),
    )(xu2, xi2, w1u, w1i, b1r, W2, b2r)


@jax.jit
def kernel(users, items, user_table, item_table, W1, b1, W2, b2):
    embed = user_table.shape[1]
    eye = jnp.eye(embed, dtype=jnp.float32)
    rows = _pack_pair(user_table.T, item_table.T, eye)
    xu2, xi2 = _sc_gather(users, items, rows, rows)
    return _tc_mlp(xu2, xi2, W1, b1, W2, b2)


# speedup vs baseline: 5.9952x; 1.0797x over previous
import functools

import jax
import jax.numpy as jnp
from jax import lax
from jax.experimental import pallas as pl
from jax.experimental.pallas import tpu as pltpu
from jax.experimental.pallas import tpu_sc as plsc

_LBLK = 16384


def _pack_body(a_ref, b_ref, eye_ref, out_ref):
    dn = (((0,), (0,)), ((), ()))
    left = lax.dot_general(
        a_ref[...], eye_ref[...], dn, preferred_element_type=jnp.float32)
    right = lax.dot_general(
        b_ref[...], eye_ref[...], dn, preferred_element_type=jnp.float32)
    out_ref[...] = jnp.concatenate([left, right], axis=1)


def _pack_pair(utab_t, itab_t, eye):
    embed, n = utab_t.shape
    grid = ((n + _LBLK - 1) // _LBLK,)
    return pl.pallas_call(
        _pack_body,
        grid=grid,
        in_specs=[
            pl.BlockSpec((embed, _LBLK), lambda g: (0, g)),
            pl.BlockSpec((embed, _LBLK), lambda g: (0, g)),
            pl.BlockSpec((embed, embed), lambda g: (0, 0)),
        ],
        out_specs=pl.BlockSpec((_LBLK, 2 * embed), lambda g: (g, 0)),
        out_shape=jax.ShapeDtypeStruct((n, 2 * embed), jnp.float32),
    )(utab_t, itab_t, eye)


def _sc_gather(users, items, urows, irows):
    info = plsc.get_sparse_core_info()
    nw = info.num_cores * info.num_subcores
    batch = users.shape[0]
    width = urows.shape[1]
    b_per_w = batch // nw
    half = b_per_w // 2

    mesh = plsc.VectorSubcoreMesh(core_axis_name="c", subcore_axis_name="s")
    out_struct = jax.ShapeDtypeStruct((batch, width), jnp.float32)

    @functools.partial(
        pl.kernel,
        mesh=mesh,
        compiler_params=pltpu.CompilerParams(needs_layout_passes=False),
        out_type=[out_struct, out_struct],
        scratch_types=[
            pltpu.VMEM((b_per_w,), jnp.int32),
            pltpu.VMEM((b_per_w,), jnp.int32),
            pltpu.VMEM((half, 128), jnp.float32),
            pltpu.VMEM((half, 128), jnp.float32),
            pltpu.SemaphoreType.DMA,
            pltpu.SemaphoreType.DMA,
            pltpu.SemaphoreType.DMA,
            pltpu.SemaphoreType.DMA,
        ],
    )
    def gather_kernel(users_hbm, items_hbm, utab_hbm, itab_hbm,
                      outu_hbm, outi_hbm,
                      idx_u, idx_i, buf_u, buf_i,
                      sem_u, sem_i, sem_wu, sem_wi):
        wid = lax.axis_index("s") * info.num_cores + lax.axis_index("c")
        base = wid * b_per_w
        pltpu.sync_copy(users_hbm.at[pl.ds(base, b_per_w)], idx_u)
        pltpu.sync_copy(items_hbm.at[pl.ds(base, b_per_w)], idx_i)

        for h in range(2):
            osl = pl.ds(base + h * half, half)
            isl = pl.ds(h * half, half)
            gu = pltpu.async_copy(
                utab_hbm.at[idx_u.at[isl]], buf_u, sem_u)
            gi = pltpu.async_copy(
                itab_hbm.at[idx_i.at[isl]], buf_i, sem_i)
            gu.wait()
            wu = pltpu.async_copy(buf_u, outu_hbm.at[osl], sem_wu)
            gi.wait()
            wi = pltpu.async_copy(buf_i, outi_hbm.at[osl], sem_wi)
            wu.wait()
            wi.wait()

    return gather_kernel(users, items, urows, irows)


def _mlp_body(xu_ref, xi_ref, w1u_ref, w1i_ref, b1_ref, w2_ref, b2_ref,
              out_ref):
    embed = w1u_ref.shape[1]
    xu = xu_ref[:, :embed]
    xi = xi_ref[:, embed:]
    dn = (((1,), (1,)), ((), ()))
    h = lax.dot_general(xu, w1u_ref[...], dn,
                        preferred_element_type=jnp.float32)
    h += lax.dot_general(xi, w1i_ref[...], dn,
                         preferred_element_type=jnp.float32)
    h = jnp.maximum(h + b1_ref[...], 0.0)
    out = jnp.sum(h * w2_ref[...], axis=1, keepdims=True)
    out_ref[...] = out + b2_ref[0, 0]


def _tc_mlp(xu2, xi2, W1, b1, W2, b2):
    batch, width = xu2.shape
    embed = width // 2
    hidden = W1.shape[0]
    w1u = W1[:, :embed]
    w1i = W1[:, embed:]
    b1r = b1.reshape(1, hidden)
    b2r = b2.reshape(1, 1)
    blk = 4096
    grid = (batch // blk,)
    return pl.pallas_call(
        _mlp_body,
        grid=grid,
        in_specs=[
            pl.BlockSpec((blk, width), lambda i: (i, 0)),
            pl.BlockSpec((blk, width), lambda i: (i, 0)),
            pl.BlockSpec((hidden, embed), lambda i: (0, 0)),
            pl.BlockSpec((hidden, embed), lambda i: (0, 0)),
            pl.BlockSpec((1, hidden), lambda i: (0, 0)),
            pl.BlockSpec((1, hidden), lambda i: (0, 0)),
            pl.BlockSpec((1, 1), lambda i: (0, 0)),
        ],
        out_specs=pl.BlockSpec((blk, 1), lambda i: (i, 0)),
        out_shape=jax.ShapeDtypeStruct((batch, 1), jnp.float32),
    )(xu2, xi2, w1u, w1i, b1r, W2, b2r)


@jax.jit
def kernel(users, items, user_table, item_table, W1, b1, W2, b2):
    embed = user_table.shape[1]
    eye = jnp.eye(embed, dtype=jnp.float32)
    rows = _pack_pair(user_table.T, item_table.T, eye)
    xu2, xi2 = _sc_gather(users, items, rows, rows)
    return _tc_mlp(xu2, xi2, W1, b1, W2, b2)


# speedup vs baseline: 6.5895x; 1.0991x over previous
import functools

import jax
import jax.numpy as jnp
from jax import lax
from jax.experimental import pallas as pl
from jax.experimental.pallas import tpu as pltpu
from jax.experimental.pallas import tpu_sc as plsc

_LBLK = 16384
_HBLK = _LBLK // 2


def _to_bf16_bits(x_f32):
    u = lax.bitcast_convert_type(x_f32, jnp.int32)
    return lax.shift_right_logical(u + jnp.int32(0x8000), 16)


def _pack_body(au_ref, ai_ref, eye_ref, out_ref):
    dn = (((0,), (0,)), ((), ()))
    at_u = lax.dot_general(
        au_ref[...], eye_ref[...], dn, preferred_element_type=jnp.float32)
    at_i = lax.dot_general(
        ai_ref[...], eye_ref[...], dn, preferred_element_type=jnp.float32)
    word = lax.shift_left(_to_bf16_bits(at_u), 16) | _to_bf16_bits(at_i)
    out_ref[...] = jnp.concatenate(
        [word[:_HBLK], word[_HBLK:]], axis=1)


def _pack_pair(utab_t, itab_t, eye):
    embed, n = utab_t.shape
    grid = ((n + _LBLK - 1) // _LBLK,)
    n_out = grid[0] * _HBLK
    return pl.pallas_call(
        _pack_body,
        grid=grid,
        in_specs=[
            pl.BlockSpec((embed, _LBLK), lambda g: (0, g)),
            pl.BlockSpec((embed, _LBLK), lambda g: (0, g)),
            pl.BlockSpec((embed, embed), lambda g: (0, 0)),
        ],
        out_specs=pl.BlockSpec((_HBLK, 2 * embed), lambda g: (g, 0)),
        out_shape=jax.ShapeDtypeStruct((n_out, 2 * embed), jnp.int32),
    )(utab_t, itab_t, eye)


def _sc_gather(ku, ki, packed):
    info = plsc.get_sparse_core_info()
    nw = info.num_cores * info.num_subcores
    batch = ku.shape[0]
    width = packed.shape[1]
    b_per_w = batch // nw
    half = b_per_w // 2

    mesh = plsc.VectorSubcoreMesh(core_axis_name="c", subcore_axis_name="s")
    out_struct = jax.ShapeDtypeStruct((batch, width), jnp.int32)

    @functools.partial(
        pl.kernel,
        mesh=mesh,
        compiler_params=pltpu.CompilerParams(needs_layout_passes=False),
        out_type=[out_struct, out_struct],
        scratch_types=[
            pltpu.VMEM((b_per_w,), jnp.int32),
            pltpu.VMEM((b_per_w,), jnp.int32),
            pltpu.VMEM((half, 128), jnp.int32),
            pltpu.VMEM((half, 128), jnp.int32),
            pltpu.SemaphoreType.DMA,
            pltpu.SemaphoreType.DMA,
            pltpu.SemaphoreType.DMA,
            pltpu.SemaphoreType.DMA,
        ],
    )
    def gather_kernel(ku_hbm, ki_hbm, tab_hbm,
                      outu_hbm, outi_hbm,
                      idx_u, idx_i, buf_u, buf_i,
                      sem_u, sem_i, sem_wu, sem_wi):
        wid = lax.axis_index("s") * info.num_cores + lax.axis_index("c")
        base = wid * b_per_w
        pltpu.sync_copy(ku_hbm.at[pl.ds(base, b_per_w)], idx_u)
        pltpu.sync_copy(ki_hbm.at[pl.ds(base, b_per_w)], idx_i)

        for h in range(2):
            osl = pl.ds(base + h * half, half)
            isl = pl.ds(h * half, half)
            gu = pltpu.async_copy(
                tab_hbm.at[idx_u.at[isl]], buf_u, sem_u)
            gi = pltpu.async_copy(
                tab_hbm.at[idx_i.at[isl]], buf_i, sem_i)
            gu.wait()
            wu = pltpu.async_copy(buf_u, outu_hbm.at[osl], sem_wu)
            gi.wait()
            wi = pltpu.async_copy(buf_i, outi_hbm.at[osl], sem_wi)
            wu.wait()
            wi.wait()

    return gather_kernel(ku, ki, packed)


def _mlp_body(xu_ref, xi_ref, mu_ref, mi_ref, w1u_ref, w1i_ref, b1_ref,
              w2_ref, b2_ref, out_ref):
    embed = w1u_ref.shape[1]
    wu = jnp.where(mu_ref[...] == 1, xu_ref[:, embed:], xu_ref[:, :embed])
    wi = jnp.where(mi_ref[...] == 1, xi_ref[:, embed:], xi_ref[:, :embed])
    xu = lax.bitcast_convert_type(wu & jnp.int32(-65536), jnp.float32)
    xi = lax.bitcast_convert_type(lax.shift_left(wi, 16), jnp.float32)
    dn = (((1,), (1,)), ((), ()))
    h = lax.dot_general(xu, w1u_ref[...], dn,
                        preferred_element_type=jnp.float32)
    h += lax.dot_general(xi, w1i_ref[...], dn,
                         preferred_element_type=jnp.float32)
    h = jnp.maximum(h + b1_ref[...], 0.0)
    out = jnp.sum(h * w2_ref[...], axis=1, keepdims=True)
    out_ref[...] = out + b2_ref[0, 0]


def _tc_mlp(xu2, xi2, mu, mi, W1, b1, W2, b2):
    batch, width = xu2.shape
    embed = width // 2
    hidden = W1.shape[0]
    w1u = W1[:, :embed]
    w1i = W1[:, embed:]
    b1r = b1.reshape(1, hidden)
    b2r = b2.reshape(1, 1)
    blk = 4096
    grid = (batch // blk,)
    return pl.pallas_call(
        _mlp_body,
        grid=grid,
        in_specs=[
            pl.BlockSpec((blk, width), lambda i: (i, 0)),
            pl.BlockSpec((blk, width), lambda i: (i, 0)),
            pl.BlockSpec((blk, embed), lambda i: (i, 0)),
            pl.BlockSpec((blk, embed), lambda i: (i, 0)),
            pl.BlockSpec((hidden, embed), lambda i: (0, 0)),
            pl.BlockSpec((hidden, embed), lambda i: (0, 0)),
            pl.BlockSpec((1, hidden), lambda i: (0, 0)),
            pl.BlockSpec((1, hidden), lambda i: (0, 0)),
            pl.BlockSpec((1, 1), lambda i: (0, 0)),
        ],
        out_specs=pl.BlockSpec((blk, 1), lambda i: (i, 0)),
        out_shape=jax.ShapeDtypeStruct((batch, 1), jnp.float32),
    )(xu2, xi2, mu, mi, w1u, w1i, b1r, W2, b2r)


def _fold_idx(i):
    return (i // _LBLK) * _HBLK + (i % _HBLK)


@jax.jit
def kernel(users, items, user_table, item_table, W1, b1, W2, b2):
    batch = users.shape[0]
    embed = user_table.shape[1]
    eye = jnp.eye(embed, dtype=jnp.float32)
    packed = _pack_pair(user_table.T, item_table.T, eye)
    ku = _fold_idx(users)
    ki = _fold_idx(items)
    mu = jnp.broadcast_to(
        ((users % _LBLK) // _HBLK).astype(jnp.int32)[:, None], (batch, embed))
    mi = jnp.broadcast_to(
        ((items % _LBLK) // _HBLK).astype(jnp.int32)[:, None], (batch, embed))
    xu2, xi2 = _sc_gather(ku, ki, packed)
    return _tc_mlp(xu2, xi2, mu, mi, W1, b1, W2, b2)
